# baseline (speedup 1.0000x reference)
import jax
import jax.numpy as jnp
from jax.experimental import pallas as pl
from jax.experimental.pallas import tpu as pltpu

_N = 100000
_D = 512
_H = 128
_K = 8
_TILE = 10000
_GRID = _N // _TILE


def _dot_t(a, b):
    return jax.lax.dot_general(a, b, (((1,), (1,)), ((), ())),
                               preferred_element_type=jnp.float32)


def _select8(vals, logs, largest):
    iota = jax.lax.broadcasted_iota(jnp.int32, vals.shape, 1)
    fill = -jnp.inf if largest else jnp.inf
    out_v = []
    out_l = []
    v = vals
    best = None
    for _ in range(_K):
        best = jnp.max(v) if largest else jnp.min(v)
        idx = jnp.min(jnp.where(v == best, iota, jnp.int32(2 ** 30)))
        onehot = iota == idx
        out_v.append(jnp.reshape(best, (1, 1)))
        out_l.append(jnp.sum(jnp.where(onehot, logs, 0.0), axis=1, keepdims=True))
        v = jnp.where(onehot, fill, v)
    return jnp.concatenate(out_v, axis=1), jnp.concatenate(out_l, axis=1), best


def _fused_kernel(feat_ul_ref, feat_ur_ref, feat_ll_ref, feat_lr_ref,
                  W_fc_ref, b_fc_ref, W_a1_ref, b_a1_ref,
                  W_a2_ref, b_a2_ref, W_ic_ref, b_ic_ref, label_ref,
                  out_m_ref, out_loss_ref,
                  m_ref, z_ref, s_ref, tv_ref, tl_ref, bv_ref, bl_ref,
                  thr_t_ref, thr_b_ref):
    i = pl.program_id(0)

    @pl.when(i == 0)
    def _init():
        m_ref[...] = jnp.full_like(m_ref, -jnp.inf)
        z_ref[...] = jnp.zeros_like(z_ref)
        s_ref[...] = jnp.zeros_like(s_ref)
        tv_ref[...] = jnp.full_like(tv_ref, -jnp.inf)
        bv_ref[...] = jnp.full_like(bv_ref, jnp.inf)
        tl_ref[...] = jnp.zeros_like(tl_ref)
        bl_ref[...] = jnp.zeros_like(bl_ref)
        thr_t_ref[0] = -jnp.inf
        thr_b_ref[0] = jnp.inf

    Wl, Wr = W_fc_ref[:, :_D // 2], W_fc_ref[:, _D // 2:]
    h_u = _dot_t(feat_ul_ref[...], Wl) + _dot_t(feat_ur_ref[...], Wr) + b_fc_ref[...]
    h_l = _dot_t(feat_ll_ref[...], Wl) + _dot_t(feat_lr_ref[...], Wr) + b_fc_ref[...]
    t_u = jnp.tanh(_dot_t(h_u, W_a1_ref[...]) + b_a1_ref[...])
    t_l = jnp.tanh(_dot_t(h_l, W_a1_ref[...]) + b_a1_ref[...])
    a_row = jnp.concatenate(
        [_dot_t(W_a2_ref[...], t_u), _dot_t(W_a2_ref[...], t_l)],
        axis=1) + b_a2_ref[...]

    tile_max = jnp.max(a_row)
    tile_min = jnp.min(a_row)

    m_old = m_ref[...]
    m_new = jnp.maximum(m_old, tile_max)
    c = jnp.exp(m_old - m_new)
    w = jnp.exp(a_row - m_new)
    m_ref[...] = m_new
    z_ref[...] = z_ref[...] * c + jnp.sum(w)
    hT = _TILE // 2
    s_ref[...] = (s_ref[...] * c
                  + jax.lax.dot_general(w[:, :hT], h_u, (((1,), (0,)), ((), ())),
                                        preferred_element_type=jnp.float32)
                  + jax.lax.dot_general(w[:, hT:], h_l, (((1,), (0,)), ((), ())),
                                        preferred_element_type=jnp.float32))

    def _l4():
        return jnp.concatenate(
            [_dot_t(W_ic_ref[...], h_u), _dot_t(W_ic_ref[...], h_l)],
            axis=1) + b_ic_ref[...]

    @pl.when(tile_max > thr_t_ref[0])
    def _merge_top():
        l4 = _l4()
        cand_v = jnp.concatenate([tv_ref[...], a_row], axis=1)
        cand_l = jnp.concatenate([tl_ref[...], l4], axis=1)
        ntv, ntl, thr = _select8(cand_v, cand_l, largest=True)
        tv_ref[...] = ntv
        tl_ref[...] = ntl
        thr_t_ref[0] = thr

    @pl.when(tile_min < thr_b_ref[0])
    def _merge_bot():
        l4 = _l4()
        cand_v = jnp.concatenate([bv_ref[...], a_row], axis=1)
        cand_l = jnp.concatenate([bl_ref[...], l4], axis=1)
        nbv, nbl, thr = _select8(cand_v, cand_l, largest=False)
        bv_ref[...] = nbv
        bl_ref[...] = nbl
        thr_b_ref[0] = thr

    @pl.when(i == _GRID - 1)
    def _finalize():
        out_m_ref[...] = s_ref[...] / z_ref[...]
        tl = tl_ref[...]
        bl = bl_ref[...]
        lab = label_ref[...]
        total = jnp.zeros((1, 1), jnp.float32)
        for cls in range(2):
            t0 = tl[2 * cls:2 * cls + 1, :]
            t1 = tl[2 * cls + 1:2 * cls + 2, :]
            mx = jnp.maximum(t0, t1)
            lse_t = mx + jnp.log(jnp.exp(t0 - mx) + jnp.exp(t1 - mx))
            b0 = bl[2 * cls:2 * cls + 1, :]
            b1 = bl[2 * cls + 1:2 * cls + 2, :]
            mxb = jnp.maximum(b0, b1)
            lse_b = mxb + jnp.log(jnp.exp(b0 - mxb) + jnp.exp(b1 - mxb))
            inst = (jnp.sum(lse_t - t1) + jnp.sum(lse_b - b0)) / (2 * _K)
            total = total + jnp.where(lab[0:1, cls:cls + 1] == 1, inst, 0.0)
        out_loss_ref[...] = total


def kernel(features, label, W_fc, b_fc, W_a1, b_a1, W_a2, b_a2, W_ic, b_ic):
    W_ic4 = W_ic.reshape(2 * 2, _H)
    b_ic4 = b_ic.reshape(2 * 2, 1)
    full = lambda shape: pl.BlockSpec(shape, lambda i: (0, 0))
    M, loss = pl.pallas_call(
        _fused_kernel,
        grid=(_GRID,),
        in_specs=[
            pl.BlockSpec((_TILE // 2, _D // 2), lambda i: (2 * i, 0)),
            pl.BlockSpec((_TILE // 2, _D // 2), lambda i: (2 * i, 1)),
            pl.BlockSpec((_TILE // 2, _D // 2), lambda i: (2 * i + 1, 0)),
            pl.BlockSpec((_TILE // 2, _D // 2), lambda i: (2 * i + 1, 1)),
            full((_H, _D)),
            full((1, _H)),
            full((_H, _H)),
            full((1, _H)),
            full((1, _H)),
            full((1, 1)),
            full((4, _H)),
            full((4, 1)),
            full((1, 2)),
        ],
        out_specs=[full((1, _H)), full((1, 1))],
        out_shape=[
            jax.ShapeDtypeStruct((1, _H), jnp.float32),
            jax.ShapeDtypeStruct((1, 1), jnp.float32),
        ],
        scratch_shapes=[
            pltpu.VMEM((1, 1), jnp.float32),
            pltpu.VMEM((1, 1), jnp.float32),
            pltpu.VMEM((1, _H), jnp.float32),
            pltpu.VMEM((1, _K), jnp.float32),
            pltpu.VMEM((4, _K), jnp.float32),
            pltpu.VMEM((1, _K), jnp.float32),
            pltpu.VMEM((4, _K), jnp.float32),
            pltpu.SMEM((1,), jnp.float32),
            pltpu.SMEM((1,), jnp.float32),
        ],
    )(features, features, features, features,
      W_fc, b_fc.reshape(1, _H), W_a1, b_a1.reshape(1, _H),
      W_a2, b_a2.reshape(1, 1), W_ic4, b_ic4, label.reshape(1, 2))
    return (M, loss[0, 0])


# speedup vs baseline: 1.0160x; 1.0160x over previous
import jax
import jax.numpy as jnp
from jax.experimental import pallas as pl
from jax.experimental.pallas import tpu as pltpu

_N = 100000
_D = 512
_H = 128
_K = 8
_TILE = 10000
_GRID = _N // _TILE


def _dot_t(a, b):
    return jax.lax.dot_general(a, b, (((1,), (1,)), ((), ())),
                               preferred_element_type=jnp.float32)


def _select8(vals, logs, largest):
    iota = jax.lax.broadcasted_iota(jnp.int32, vals.shape, 1)
    fill = -jnp.inf if largest else jnp.inf
    out_v = []
    out_l = []
    v = vals
    best = None
    for _ in range(_K):
        best = jnp.max(v) if largest else jnp.min(v)
        idx = jnp.min(jnp.where(v == best, iota, jnp.int32(2 ** 30)))
        onehot = iota == idx
        out_v.append(jnp.reshape(best, (1, 1)))
        out_l.append(jnp.sum(jnp.where(onehot, logs, 0.0), axis=1, keepdims=True))
        v = jnp.where(onehot, fill, v)
    return jnp.concatenate(out_v, axis=1), jnp.concatenate(out_l, axis=1), best


def _fused_kernel(feat_u_ref, feat_l_ref,
                  W_fc_ref, b_fc_ref, W_a1_ref, b_a1_ref,
                  W_a2_ref, b_a2_ref, W_ic_ref, b_ic_ref, label_ref,
                  out_m_ref, out_loss_ref,
                  m_ref, z_ref, s_ref, tv_ref, tl_ref, bv_ref, bl_ref,
                  thr_t_ref, thr_b_ref):
    i = pl.program_id(0)

    @pl.when(i == 0)
    def _init():
        m_ref[...] = jnp.full_like(m_ref, -jnp.inf)
        z_ref[...] = jnp.zeros_like(z_ref)
        s_ref[...] = jnp.zeros_like(s_ref)
        tv_ref[...] = jnp.full_like(tv_ref, -jnp.inf)
        bv_ref[...] = jnp.full_like(bv_ref, jnp.inf)
        tl_ref[...] = jnp.zeros_like(tl_ref)
        bl_ref[...] = jnp.zeros_like(bl_ref)
        thr_t_ref[0] = -jnp.inf
        thr_b_ref[0] = jnp.inf

    h_u = _dot_t(feat_u_ref[...], W_fc_ref[...]) + b_fc_ref[...]
    h_l = _dot_t(feat_l_ref[...], W_fc_ref[...]) + b_fc_ref[...]
    t_u = jnp.tanh(_dot_t(h_u, W_a1_ref[...]) + b_a1_ref[...])
    t_l = jnp.tanh(_dot_t(h_l, W_a1_ref[...]) + b_a1_ref[...])
    a_row = jnp.concatenate(
        [_dot_t(W_a2_ref[...], t_u), _dot_t(W_a2_ref[...], t_l)],
        axis=1) + b_a2_ref[...]

    tile_max = jnp.max(a_row)
    tile_min = jnp.min(a_row)

    m_old = m_ref[...]
    m_new = jnp.maximum(m_old, tile_max)
    c = jnp.exp(m_old - m_new)
    w = jnp.exp(a_row - m_new)
    m_ref[...] = m_new
    z_ref[...] = z_ref[...] * c + jnp.sum(w)
    hT = _TILE // 2
    s_ref[...] = (s_ref[...] * c
                  + jax.lax.dot_general(w[:, :hT], h_u, (((1,), (0,)), ((), ())),
                                        preferred_element_type=jnp.float32)
                  + jax.lax.dot_general(w[:, hT:], h_l, (((1,), (0,)), ((), ())),
                                        preferred_element_type=jnp.float32))

    def _l4():
        return jnp.concatenate(
            [_dot_t(W_ic_ref[...], h_u), _dot_t(W_ic_ref[...], h_l)],
            axis=1) + b_ic_ref[...]

    @pl.when(tile_max > thr_t_ref[0])
    def _merge_top():
        l4 = _l4()
        cand_v = jnp.concatenate([tv_ref[...], a_row], axis=1)
        cand_l = jnp.concatenate([tl_ref[...], l4], axis=1)
        ntv, ntl, thr = _select8(cand_v, cand_l, largest=True)
        tv_ref[...] = ntv
        tl_ref[...] = ntl
        thr_t_ref[0] = thr

    @pl.when(tile_min < thr_b_ref[0])
    def _merge_bot():
        l4 = _l4()
        cand_v = jnp.concatenate([bv_ref[...], a_row], axis=1)
        cand_l = jnp.concatenate([bl_ref[...], l4], axis=1)
        nbv, nbl, thr = _select8(cand_v, cand_l, largest=False)
        bv_ref[...] = nbv
        bl_ref[...] = nbl
        thr_b_ref[0] = thr

    @pl.when(i == _GRID - 1)
    def _finalize():
        out_m_ref[...] = s_ref[...] / z_ref[...]
        tl = tl_ref[...]
        bl = bl_ref[...]
        lab = label_ref[...]
        total = jnp.zeros((1, 1), jnp.float32)
        for cls in range(2):
            t0 = tl[2 * cls:2 * cls + 1, :]
            t1 = tl[2 * cls + 1:2 * cls + 2, :]
            mx = jnp.maximum(t0, t1)
            lse_t = mx + jnp.log(jnp.exp(t0 - mx) + jnp.exp(t1 - mx))
            b0 = bl[2 * cls:2 * cls + 1, :]
            b1 = bl[2 * cls + 1:2 * cls + 2, :]
            mxb = jnp.maximum(b0, b1)
            lse_b = mxb + jnp.log(jnp.exp(b0 - mxb) + jnp.exp(b1 - mxb))
            inst = (jnp.sum(lse_t - t1) + jnp.sum(lse_b - b0)) / (2 * _K)
            total = total + jnp.where(lab[0:1, cls:cls + 1] == 1, inst, 0.0)
        out_loss_ref[...] = total


def kernel(features, label, W_fc, b_fc, W_a1, b_a1, W_a2, b_a2, W_ic, b_ic):
    W_ic4 = W_ic.reshape(2 * 2, _H)
    b_ic4 = b_ic.reshape(2 * 2, 1)
    full = lambda shape: pl.BlockSpec(shape, lambda i: (0, 0))
    M, loss = pl.pallas_call(
        _fused_kernel,
        grid=(_GRID,),
        in_specs=[
            pl.BlockSpec((_TILE // 2, _D), lambda i: (2 * i, 0)),
            pl.BlockSpec((_TILE // 2, _D), lambda i: (2 * i + 1, 0)),
            full((_H, _D)),
            full((1, _H)),
            full((_H, _H)),
            full((1, _H)),
            full((1, _H)),
            full((1, 1)),
            full((4, _H)),
            full((4, 1)),
            full((1, 2)),
        ],
        out_specs=[full((1, _H)), full((1, 1))],
        out_shape=[
            jax.ShapeDtypeStruct((1, _H), jnp.float32),
            jax.ShapeDtypeStruct((1, 1), jnp.float32),
        ],
        scratch_shapes=[
            pltpu.VMEM((1, 1), jnp.float32),
            pltpu.VMEM((1, 1), jnp.float32),
            pltpu.VMEM((1, _H), jnp.float32),
            pltpu.VMEM((1, _K), jnp.float32),
            pltpu.VMEM((4, _K), jnp.float32),
            pltpu.VMEM((1, _K), jnp.float32),
            pltpu.VMEM((4, _K), jnp.float32),
            pltpu.SMEM((1,), jnp.float32),
            pltpu.SMEM((1,), jnp.float32),
        ],
    )(features, features,
      W_fc, b_fc.reshape(1, _H), W_a1, b_a1.reshape(1, _H),
      W_a2, b_a2.reshape(1, 1), W_ic4, b_ic4, label.reshape(1, 2))
    return (M, loss[0, 0])


# speedup vs baseline: 1.3646x; 1.3430x over previous
import jax
import jax.numpy as jnp
from jax.experimental import pallas as pl
from jax.experimental.pallas import tpu as pltpu

_N = 100000
_D = 512
_H = 128
_K = 8
_TILE = 5000
_GRID = _N // _TILE


def _dot_t(a, b):
    return jax.lax.dot_general(a, b, (((1,), (1,)), ((), ())),
                               preferred_element_type=jnp.float32)


def _stream_kernel(feat_a_ref, feat_b_ref, W_fc_ref, b_fc_ref, W_a1_ref,
                   b_a1_ref, W_a2_ref, b_a2_ref, W_ic_ref, b_ic_ref,
                   a_ref, l4_ref, m_ref, z_ref, s_ref):
    h = (_dot_t(feat_a_ref[...], W_fc_ref[:, :_D // 2]) +
         _dot_t(feat_b_ref[...], W_fc_ref[:, _D // 2:]) + b_fc_ref[...])
    t = jnp.tanh(_dot_t(h, W_a1_ref[...]) + b_a1_ref[...])
    a_row = _dot_t(W_a2_ref[...], t) + b_a2_ref[...]
    l4_ref[...] = (_dot_t(W_ic_ref[...], h) + b_ic_ref[...]).reshape(1, 4, _TILE)
    a_ref[...] = a_row.reshape(1, 1, _TILE)
    m_i = jnp.max(a_row)
    w = jnp.exp(a_row - m_i)
    m_ref[...] = jnp.full((1, 1, 1), m_i)
    z_ref[...] = jnp.sum(w).reshape(1, 1, 1)
    s_ref[...] = jax.lax.dot_general(
        w, h, (((1,), (0,)), ((), ())),
        preferred_element_type=jnp.float32).reshape(1, 1, _H)


def _finalize_kernel(a_ref, l4_ref, m_ref, z_ref, s_ref, label_ref,
                     out_m_ref, out_loss_ref):
    m = m_ref[...]
    gm = jnp.max(m)
    scale = jnp.exp(m - gm)
    Z = jnp.sum(z_ref[...] * scale)
    out_m_ref[...] = jnp.sum(s_ref[...] * scale, axis=0) / Z

    a2 = a_ref[...].reshape(_GRID, _TILE)
    iota = (jax.lax.broadcasted_iota(jnp.int32, a2.shape, 0) * _TILE +
            jax.lax.broadcasted_iota(jnp.int32, a2.shape, 1))

    def select(v, largest):
        fill = -jnp.inf if largest else jnp.inf
        logits = []
        for _ in range(_K):
            best = jnp.max(v) if largest else jnp.min(v)
            idx = jnp.min(jnp.where(v == best, iota, jnp.int32(2 ** 30)))
            onehot = iota == idx
            logits.append([jnp.sum(jnp.where(onehot, l4_ref[:, j, :], 0.0))
                           for j in range(4)])
            v = jnp.where(onehot, fill, v)
        return logits

    top_l = select(a2, largest=True)
    bot_l = select(a2, largest=False)

    lab = label_ref[...]
    total = jnp.zeros((1, 1), jnp.float32)
    for cls in range(2):
        acc = jnp.float32(0.0)
        for l in top_l:
            l0, l1 = l[2 * cls], l[2 * cls + 1]
            mx = jnp.maximum(l0, l1)
            acc = acc + (mx + jnp.log(jnp.exp(l0 - mx) + jnp.exp(l1 - mx)) - l1)
        for l in bot_l:
            l0, l1 = l[2 * cls], l[2 * cls + 1]
            mx = jnp.maximum(l0, l1)
            acc = acc + (mx + jnp.log(jnp.exp(l0 - mx) + jnp.exp(l1 - mx)) - l0)
        inst = jnp.reshape(acc / (2 * _K), (1, 1))
        total = total + jnp.where(lab[0:1, cls:cls + 1] == 1, inst, 0.0)
    out_loss_ref[...] = total


def kernel(features, label, W_fc, b_fc, W_a1, b_a1, W_a2, b_a2, W_ic, b_ic):
    W_ic4 = W_ic.reshape(2 * 2, _H)
    b_ic4 = b_ic.reshape(2 * 2, 1)
    full2 = lambda shape: pl.BlockSpec(shape, lambda i: (0, 0))
    a, l4, m, z, s = pl.pallas_call(
        _stream_kernel,
        grid=(_GRID,),
        in_specs=[
            pl.BlockSpec((_TILE, _D // 2), lambda i: (i, 0)),
            pl.BlockSpec((_TILE, _D // 2), lambda i: (i, 1)),
            full2((_H, _D)),
            full2((1, _H)),
            full2((_H, _H)),
            full2((1, _H)),
            full2((1, _H)),
            full2((1, 1)),
            full2((4, _H)),
            full2((4, 1)),
        ],
        out_specs=[
            pl.BlockSpec((1, 1, _TILE), lambda i: (i, 0, 0)),
            pl.BlockSpec((1, 4, _TILE), lambda i: (i, 0, 0)),
            pl.BlockSpec((1, 1, 1), lambda i: (i, 0, 0)),
            pl.BlockSpec((1, 1, 1), lambda i: (i, 0, 0)),
            pl.BlockSpec((1, 1, _H), lambda i: (i, 0, 0)),
        ],
        out_shape=[
            jax.ShapeDtypeStruct((_GRID, 1, _TILE), jnp.float32),
            jax.ShapeDtypeStruct((_GRID, 4, _TILE), jnp.float32),
            jax.ShapeDtypeStruct((_GRID, 1, 1), jnp.float32),
            jax.ShapeDtypeStruct((_GRID, 1, 1), jnp.float32),
            jax.ShapeDtypeStruct((_GRID, 1, _H), jnp.float32),
        ],
        compiler_params=pltpu.CompilerParams(
            dimension_semantics=("parallel",)),
    )(features, features, W_fc, b_fc.reshape(1, _H), W_a1,
      b_a1.reshape(1, _H), W_a2, b_a2.reshape(1, 1), W_ic4, b_ic4)

    M, loss = pl.pallas_call(
        _finalize_kernel,
        out_shape=[
            jax.ShapeDtypeStruct((1, _H), jnp.float32),
            jax.ShapeDtypeStruct((1, 1), jnp.float32),
        ],
    )(a, l4, m, z, s, label.reshape(1, 2))
    return (M, loss[0, 0])


# speedup vs baseline: 1.3750x; 1.0076x over previous
import jax
import jax.numpy as jnp
from jax.experimental import pallas as pl
from jax.experimental.pallas import tpu as pltpu

_N = 100000
_D = 512
_H = 128
_K = 8
_TILE = 5000
_GRID = _N // _TILE


def _dot_t(a, b):
    return jax.lax.dot_general(a, b, (((1,), (1,)), ((), ())),
                               preferred_element_type=jnp.float32)


def _stream_kernel(feat_a_ref, feat_b_ref, W_fc_ref, b_fc_ref, W_a1_ref,
                   b_a1_ref, W_a2_ref, b_a2_ref, W_ic_ref, b_ic_ref,
                   a_ref, l4_ref, m_ref, z_ref, s_ref):
    h = (_dot_t(feat_a_ref[...], W_fc_ref[:, :_D // 2]) +
         _dot_t(feat_b_ref[...], W_fc_ref[:, _D // 2:]) + b_fc_ref[...])
    t = jnp.tanh(_dot_t(h, W_a1_ref[...]) + b_a1_ref[...])
    a_row = _dot_t(W_a2_ref[...], t) + b_a2_ref[...]
    l4_ref[...] = (_dot_t(W_ic_ref[...], h) + b_ic_ref[...]).reshape(1, 4, _TILE)
    a_ref[...] = a_row.reshape(1, 1, _TILE)
    m_i = jnp.max(a_row)
    w = jnp.exp(a_row - m_i)
    m_ref[...] = jnp.full((1, 1, 1), m_i)
    z_ref[...] = jnp.sum(w).reshape(1, 1, 1)
    s_ref[...] = jax.lax.dot_general(
        w, h, (((1,), (0,)), ((), ())),
        preferred_element_type=jnp.float32).reshape(1, 1, _H)


def _finalize_kernel(a_ref, l4_ref, m_ref, z_ref, s_ref, label_ref,
                     out_m_ref, out_loss_ref):
    m = m_ref[...]
    gm = jnp.max(m)
    scale = jnp.exp(m - gm)
    Z = jnp.sum(z_ref[...] * scale)
    out_m_ref[...] = jnp.sum(s_ref[...] * scale, axis=0) / Z

    a2 = a_ref[...].reshape(_GRID, _TILE)
    lane = jax.lax.broadcasted_iota(jnp.int32, (1, _TILE), 1)
    gidx = (jax.lax.broadcasted_iota(jnp.int32, a2.shape, 0) * _TILE +
            jax.lax.broadcasted_iota(jnp.int32, a2.shape, 1)).astype(jnp.float32)

    def pick_cols(cv, largest):
        fill = -jnp.inf if largest else jnp.inf
        rows = []
        v = cv
        for _ in range(_K):
            best = jnp.max(v) if largest else jnp.min(v)
            idx = jnp.min(jnp.where(v == best, lane, jnp.int32(2 ** 30)))
            oh = lane == idx
            rows.append(oh.astype(jnp.float32))
            v = jnp.where(oh, fill, v)
        return rows

    colmax = jnp.max(a2, axis=0, keepdims=True)
    colmin = jnp.min(a2, axis=0, keepdims=True)
    sel = jnp.concatenate(pick_cols(colmax, True) + pick_cols(colmin, False),
                          axis=0)

    def _compact(x):
        return jax.lax.dot_general(x, sel, (((1,), (1,)), ((), ())),
                                   preferred_element_type=jnp.float32)

    ca = _compact(a2)
    ci = _compact(gidx)
    cl = [_compact(l4_ref[:, j, :]) for j in range(4)]

    def select_small(cols, largest):
        fill = -jnp.inf if largest else jnp.inf
        v = ca[:, cols]
        ic = ci[:, cols]
        l4c = [c[:, cols] for c in cl]
        logits = []
        for _ in range(_K):
            best = jnp.max(v) if largest else jnp.min(v)
            eq = v == best
            first = jnp.min(jnp.where(eq, ic, jnp.float32(3e9)))
            oh = jnp.logical_and(eq, ic == first)
            logits.append([jnp.sum(jnp.where(oh, c, 0.0)) for c in l4c])
            v = jnp.where(oh, fill, v)
        return logits

    top_l = select_small(slice(0, _K), largest=True)
    bot_l = select_small(slice(_K, 2 * _K), largest=False)

    lab = label_ref[...]
    total = jnp.zeros((1, 1), jnp.float32)
    for cls in range(2):
        acc = jnp.float32(0.0)
        for l in top_l:
            l0, l1 = l[2 * cls], l[2 * cls + 1]
            mx = jnp.maximum(l0, l1)
            acc = acc + (mx + jnp.log(jnp.exp(l0 - mx) + jnp.exp(l1 - mx)) - l1)
        for l in bot_l:
            l0, l1 = l[2 * cls], l[2 * cls + 1]
            mx = jnp.maximum(l0, l1)
            acc = acc + (mx + jnp.log(jnp.exp(l0 - mx) + jnp.exp(l1 - mx)) - l0)
        inst = jnp.reshape(acc / (2 * _K), (1, 1))
        total = total + jnp.where(lab[0:1, cls:cls + 1] == 1, inst, 0.0)
    out_loss_ref[...] = total


def kernel(features, label, W_fc, b_fc, W_a1, b_a1, W_a2, b_a2, W_ic, b_ic):
    W_ic4 = W_ic.reshape(2 * 2, _H)
    b_ic4 = b_ic.reshape(2 * 2, 1)
    full2 = lambda shape: pl.BlockSpec(shape, lambda i: (0, 0))
    a, l4, m, z, s = pl.pallas_call(
        _stream_kernel,
        grid=(_GRID,),
        in_specs=[
            pl.BlockSpec((_TILE, _D // 2), lambda i: (i, 0)),
            pl.BlockSpec((_TILE, _D // 2), lambda i: (i, 1)),
            full2((_H, _D)),
            full2((1, _H)),
            full2((_H, _H)),
            full2((1, _H)),
            full2((1, _H)),
            full2((1, 1)),
            full2((4, _H)),
            full2((4, 1)),
        ],
        out_specs=[
            pl.BlockSpec((1, 1, _TILE), lambda i: (i, 0, 0)),
            pl.BlockSpec((1, 4, _TILE), lambda i: (i, 0, 0)),
            pl.BlockSpec((1, 1, 1), lambda i: (i, 0, 0)),
            pl.BlockSpec((1, 1, 1), lambda i: (i, 0, 0)),
            pl.BlockSpec((1, 1, _H), lambda i: (i, 0, 0)),
        ],
        out_shape=[
            jax.ShapeDtypeStruct((_GRID, 1, _TILE), jnp.float32),
            jax.ShapeDtypeStruct((_GRID, 4, _TILE), jnp.float32),
            jax.ShapeDtypeStruct((_GRID, 1, 1), jnp.float32),
            jax.ShapeDtypeStruct((_GRID, 1, 1), jnp.float32),
            jax.ShapeDtypeStruct((_GRID, 1, _H), jnp.float32),
        ],
        compiler_params=pltpu.CompilerParams(
            dimension_semantics=("parallel",)),
    )(features, features, W_fc, b_fc.reshape(1, _H), W_a1,
      b_a1.reshape(1, _H), W_a2, b_a2.reshape(1, 1), W_ic4, b_ic4)

    M, loss = pl.pallas_call(
        _finalize_kernel,
        out_shape=[
            jax.ShapeDtypeStruct((1, _H), jnp.float32),
            jax.ShapeDtypeStruct((1, 1), jnp.float32),
        ],
    )(a, l4, m, z, s, label.reshape(1, 2))
    return (M, loss[0, 0])
